# Initial kernel scaffold; baseline (speedup 1.0000x reference)
import jax
import jax.numpy as jnp
from jax.experimental import pallas as pl


def kernel(x_nchw, e0_w, e1_w, e1_g, e1_b, e2_w, e2_g, e2_b, e3_w, e3_g, e3_b, e4_w, e4_g, e4_b, e5_w, e5_g, e5_b, d4_w1, d4_g1, d4_b1, d4_w2, d4_g2, d4_b2, d3_w1, d3_g1, d3_b1, d3_w2, d3_g2, d3_b2, d2_w1, d2_g1, d2_b1, d2_w2, d2_g2, d2_b2, d1_w1, d1_g1, d1_b1, d1_w2, d1_g2, d1_b2, d0_w1, d0_g1, d0_b1, d0_w2, d0_g2, d0_b2, final_w, final_b):
    raise NotImplementedError("write your pallas kernel here")


# speedup vs baseline: 1.1779x; 1.1779x over previous
import functools

import jax
import jax.numpy as jnp
from jax import lax
from jax.experimental import pallas as pl
from jax.experimental.pallas import tpu as pltpu

EPS = 1e-5
VMEM_BUDGET = 20 * 1024 * 1024
VMEM_LIMIT = 48 * 1024 * 1024


def _ceil_to(x, m):
    return ((x + m - 1) // m) * m


def _pad_rows(a, mult):
    M = a.shape[0]
    Mp = _ceil_to(M, mult)
    if Mp != M:
        a = jnp.pad(a, ((0, Mp - M), (0, 0)))
    return a, Mp


def _pick_tm(M, K, N, in_bytes=2):
    fixed = 2 * K * N * in_bytes + 2 * 8 * N * 4
    per_row = 2 * K * in_bytes + 2 * N * in_bytes + N * 4
    tm = (VMEM_BUDGET - fixed) // max(per_row, 1)
    tm = int(max(8, min(4096, tm)))
    tm -= tm % 8
    Mp8 = _ceil_to(M, 8)
    if Mp8 <= tm:
        return Mp8
    return tm


_CPARAMS = pltpu.CompilerParams(dimension_semantics=("parallel",),
                                vmem_limit_bytes=VMEM_LIMIT)


def _gemm_stats_kernel(a_ref, w_ref, y_ref, st_ref):
    y = jnp.dot(a_ref[...], w_ref[...], preferred_element_type=jnp.float32)
    y_ref[...] = y.astype(y_ref.dtype)
    st_ref[0:1, :] = jnp.sum(y, axis=0, keepdims=True)
    st_ref[1:2, :] = jnp.sum(y * y, axis=0, keepdims=True)
    st_ref[2:8, :] = jnp.zeros((6, y.shape[1]), jnp.float32)


def _affine_relu_kernel(y_ref, s_ref, t_ref, o_ref):
    o = y_ref[...].astype(jnp.float32) * s_ref[...] + t_ref[...]
    o_ref[...] = jnp.maximum(o, 0.0).astype(o_ref.dtype)


def _matmul_kernel(a_ref, w_ref, o_ref):
    o_ref[...] = jnp.dot(a_ref[...], w_ref[...],
                         preferred_element_type=jnp.float32).astype(o_ref.dtype)


def _matmul_bias_kernel(a_ref, w_ref, b_ref, o_ref):
    o_ref[...] = (jnp.dot(a_ref[...], w_ref[...],
                          preferred_element_type=jnp.float32)
                  + b_ref[...]).astype(o_ref.dtype)


def gemm_bn_relu(a, w, gamma, beta, n_valid_rows, dtype=jnp.bfloat16):
    a = a.astype(dtype)
    w = w.astype(dtype)
    M, K = a.shape
    N = w.shape[1]
    tm = _pick_tm(M, K, N, in_bytes=4)
    a, Mp = _pad_rows(a, tm)
    nt = Mp // tm

    y, st = pl.pallas_call(
        _gemm_stats_kernel,
        grid=(nt,),
        in_specs=[
            pl.BlockSpec((tm, K), lambda i: (i, 0)),
            pl.BlockSpec((K, N), lambda i: (0, 0)),
        ],
        out_specs=(
            pl.BlockSpec((tm, N), lambda i: (i, 0)),
            pl.BlockSpec((8, N), lambda i: (i, 0)),
        ),
        out_shape=(
            jax.ShapeDtypeStruct((Mp, N), jnp.float32),
            jax.ShapeDtypeStruct((8 * nt, N), jnp.float32),
        ),
        compiler_params=_CPARAMS,
    )(a, w)

    inv_m = 1.0 / float(n_valid_rows)
    st = st.reshape(nt, 8, N)
    mean = jnp.sum(st[:, 0, :], axis=0) * inv_m
    ex2 = jnp.sum(st[:, 1, :], axis=0) * inv_m
    var = jnp.maximum(ex2 - mean * mean, 0.0)
    scale = gamma * lax.rsqrt(var + EPS)
    shift = beta - mean * scale

    out = pl.pallas_call(
        _affine_relu_kernel,
        grid=(nt,),
        in_specs=[
            pl.BlockSpec((tm, N), lambda i: (i, 0)),
            pl.BlockSpec((1, N), lambda i: (0, 0)),
            pl.BlockSpec((1, N), lambda i: (0, 0)),
        ],
        out_specs=pl.BlockSpec((tm, N), lambda i: (i, 0)),
        out_shape=jax.ShapeDtypeStruct((Mp, N), dtype),
        compiler_params=_CPARAMS,
    )(y, scale.reshape(1, N), shift.reshape(1, N))
    return out[:M]


def pallas_matmul(a, w, bias=None, out_dtype=jnp.bfloat16, dtype=jnp.bfloat16):
    a = a.astype(dtype)
    w = w.astype(dtype)
    M, K = a.shape
    N = w.shape[1]
    tm = _pick_tm(M, K, N, in_bytes=a.dtype.itemsize)
    a, Mp = _pad_rows(a, tm)
    nt = Mp // tm

    in_specs = [pl.BlockSpec((tm, K), lambda i: (i, 0)),
                pl.BlockSpec((K, N), lambda i: (0, 0))]
    args = [a, w]
    kern = _matmul_kernel
    if bias is not None:
        in_specs.append(pl.BlockSpec((1, N), lambda i: (0, 0)))
        args.append(bias.reshape(1, N).astype(jnp.float32))
        kern = _matmul_bias_kernel

    out = pl.pallas_call(
        kern,
        grid=(nt,),
        in_specs=in_specs,
        out_specs=pl.BlockSpec((tm, N), lambda i: (i, 0)),
        out_shape=jax.ShapeDtypeStruct((Mp, N), out_dtype),
        compiler_params=_CPARAMS,
    )(*args)
    return out[:M]


def im2col(x, kh, kw, stride, pad):
    B, H, W, C = x.shape
    xp = jnp.pad(x, ((0, 0), (pad, pad), (pad, pad), (0, 0)))
    Ho = (H + 2 * pad - kh) // stride + 1
    Wo = (W + 2 * pad - kw) // stride + 1
    cols = []
    for i in range(kh):
        for j in range(kw):
            cols.append(xp[:, i:i + Ho * stride:stride, j:j + Wo * stride:stride, :])
    patches = jnp.stack(cols, axis=3)
    return patches.reshape(B * Ho * Wo, kh * kw * C), (Ho, Wo)


def _w_to_gemm(w):
    Cout, Cin, kh, kw = w.shape
    return jnp.transpose(w, (2, 3, 1, 0)).reshape(kh * kw * Cin, Cout)


def conv_bn_relu(x, w, gamma, beta, stride=1, pad=1, dtype=jnp.bfloat16):
    B = x.shape[0]
    Cout = w.shape[0]
    a, (Ho, Wo) = im2col(x.astype(dtype), w.shape[2], w.shape[3], stride, pad)
    y = gemm_bn_relu(a, _w_to_gemm(w), gamma, beta,
                     n_valid_rows=a.shape[0], dtype=dtype)
    return y.reshape(B, Ho, Wo, Cout)


def conv_plain(x, w, stride, pad, dtype=jnp.bfloat16):
    B = x.shape[0]
    Cout = w.shape[0]
    a, (Ho, Wo) = im2col(x.astype(dtype), w.shape[2], w.shape[3], stride, pad)
    y = pallas_matmul(a, _w_to_gemm(w), out_dtype=dtype, dtype=dtype)
    return y.reshape(B, Ho, Wo, Cout)


def _interp_matrix(n):
    m = 2 * n
    if n == 1:
        return jnp.ones((m, 1), jnp.float32)
    src = jnp.arange(m, dtype=jnp.float32) * (n - 1) / (m - 1)
    i0 = jnp.clip(jnp.floor(src), 0, n - 2).astype(jnp.int32)
    frac = src - i0.astype(jnp.float32)
    r = jnp.zeros((m, n), jnp.float32)
    r = r.at[jnp.arange(m), i0].add(1.0 - frac)
    r = r.at[jnp.arange(m), i0 + 1].add(frac)
    return r


def _up_matrix(n, times):
    r = _interp_matrix(n)
    for _ in range(times - 1):
        n = 2 * n
        r = _interp_matrix(n) @ r
    return r


def bilinear_up(x, times=1, out_dtype=jnp.bfloat16, dtype=jnp.bfloat16):
    B, H, W, C = x.shape
    rh_t = _up_matrix(H, times).T
    rw_t = _up_matrix(W, times).T
    Ho, Wo = H << times, W << times
    xr = jnp.transpose(x, (0, 2, 3, 1)).reshape(B * W * C, H)
    y1 = pallas_matmul(xr, rh_t, out_dtype=dtype, dtype=dtype).reshape(B, W, C, Ho)
    y1 = jnp.transpose(y1, (0, 3, 2, 1)).reshape(B * Ho * C, W)
    y2 = pallas_matmul(y1, rw_t, out_dtype=out_dtype,
                       dtype=dtype).reshape(B, Ho, C, Wo)
    return jnp.transpose(y2, (0, 1, 3, 2))


def _vgg_block(x, w1, g1, b1, w2, g2, b2, dtype=jnp.bfloat16):
    x = conv_bn_relu(x, w1, g1, b1, 1, 1, dtype=dtype)
    return conv_bn_relu(x, w2, g2, b2, 1, 1, dtype=dtype)


def kernel(x_nchw, e0_w, e1_w, e1_g, e1_b, e2_w, e2_g, e2_b, e3_w, e3_g, e3_b,
           e4_w, e4_g, e4_b, e5_w, e5_g, e5_b,
           d4_w1, d4_g1, d4_b1, d4_w2, d4_g2, d4_b2,
           d3_w1, d3_g1, d3_b1, d3_w2, d3_g2, d3_b2,
           d2_w1, d2_g1, d2_b1, d2_w2, d2_g2, d2_b2,
           d1_w1, d1_g1, d1_b1, d1_w2, d1_g2, d1_b2,
           d0_w1, d0_g1, d0_b1, d0_w2, d0_g2, d0_b2,
           final_w, final_b):
    x = jnp.transpose(x_nchw, (0, 2, 3, 1)).astype(jnp.float32)
    f32 = jnp.float32

    x0_0 = conv_plain(x, e0_w, 2, 1, dtype=f32)
    x1_0 = conv_bn_relu(x0_0, e1_w, e1_g, e1_b, 2, 1, dtype=f32)
    x2_0 = conv_bn_relu(x1_0, e2_w, e2_g, e2_b, 4, 1, dtype=f32)
    x3_0 = conv_bn_relu(x2_0, e3_w, e3_g, e3_b, 1, 1, dtype=f32)
    x4_0 = conv_bn_relu(x3_0, e4_w, e4_g, e4_b, 2, 1, dtype=f32)
    x5_0 = conv_bn_relu(x4_0, e5_w, e5_g, e5_b, 1, 1, dtype=f32)

    x4_1 = _vgg_block(jnp.concatenate([x4_0, x5_0], -1),
                      d4_w1, d4_g1, d4_b1, d4_w2, d4_g2, d4_b2, dtype=f32)
    x3_2 = _vgg_block(jnp.concatenate([x3_0, bilinear_up(x4_1, dtype=f32,
                                                         out_dtype=f32)], -1),
                      d3_w1, d3_g1, d3_b1, d3_w2, d3_g2, d3_b2, dtype=f32)
    x2_3 = _vgg_block(jnp.concatenate([x2_0, x3_2], -1),
                      d2_w1, d2_g1, d2_b1, d2_w2, d2_g2, d2_b2, dtype=f32)
    x1_4 = _vgg_block(jnp.concatenate([x1_0,
                                       bilinear_up(x2_3, times=2, dtype=f32,
                                                   out_dtype=f32)], -1),
                      d1_w1, d1_g1, d1_b1, d1_w2, d1_g2, d1_b2)
    x0_5 = _vgg_block(jnp.concatenate([x0_0.astype(jnp.bfloat16),
                                       bilinear_up(x1_4, dtype=f32)], -1),
                      d0_w1, d0_g1, d0_b1, d0_w2, d0_g2, d0_b2)

    B, H, W, _ = x0_5.shape
    wm = final_w[:, :, 0, 0].T
    y = pallas_matmul(x0_5.reshape(B * H * W, -1), wm,
                      bias=final_b).reshape(B, H, W, -1)
    out = bilinear_up(y, out_dtype=jnp.float32)
    return jnp.transpose(out, (0, 3, 1, 2))
